# initial kernel scaffold (unmeasured)
import jax
import jax.numpy as jnp
from jax import lax
from jax.experimental import pallas as pl
from jax.experimental.pallas import tpu as pltpu

N_DEV = 4


def kernel(A, B):
    M = A.shape[0]
    N = B.shape[1]
    CH = M // N_DEV

    A16 = A.astype(jnp.bfloat16)
    B16 = B.astype(jnp.bfloat16)

    def body(a_ref, b_ref, out_ref, rs_ref, ag_ref,
             rs_send, rs_recv, ag_send, ag_recv):
        my = lax.axis_index("i")
        right = lax.rem(my + 1, N_DEV)
        left = lax.rem(my + N_DEV - 1, N_DEV)

        barrier = pltpu.get_barrier_semaphore()
        for nbr in (left, right):
            pl.semaphore_signal(barrier, inc=1, device_id=(nbr,),
                                device_id_type=pl.DeviceIdType.MESH)
        pl.semaphore_wait(barrier, 2)

        def partial_chunk(c):
            a_blk = a_ref[pl.ds(c * CH, CH), :]
            return jnp.dot(a_blk, b_ref[...],
                           preferred_element_type=jnp.float32)

        rs_ref[0] = partial_chunk(my).astype(jnp.bfloat16)

        acc = None
        for s in range(N_DEV - 1):
            rdma = pltpu.make_async_remote_copy(
                src_ref=rs_ref.at[s % 2],
                dst_ref=rs_ref.at[(s + 1) % 2],
                send_sem=rs_send.at[s],
                recv_sem=rs_recv.at[s],
                device_id=(right,),
                device_id_type=pl.DeviceIdType.MESH,
            )
            rdma.start()
            rdma.wait()
            c = lax.rem(my - s - 1 + N_DEV, N_DEV)
            acc = rs_ref[(s + 1) % 2].astype(jnp.float32) + partial_chunk(c)
            if s < N_DEV - 2:
                rs_ref[(s + 1) % 2] = acc.astype(jnp.bfloat16)

        own = lax.rem(my + 1, N_DEV)
        own_bf16 = acc.astype(jnp.bfloat16)
        out_ref[pl.ds(own * CH, CH), :] = own_bf16
        ag_ref[0] = own_bf16

        for h in range(N_DEV - 1):
            rdma = pltpu.make_async_remote_copy(
                src_ref=ag_ref.at[h % 2],
                dst_ref=ag_ref.at[(h + 1) % 2],
                send_sem=ag_send.at[h],
                recv_sem=ag_recv.at[h],
                device_id=(right,),
                device_id_type=pl.DeviceIdType.MESH,
            )
            rdma.start()
            rdma.wait()
            c = lax.rem(my - h + N_DEV, N_DEV)
            out_ref[pl.ds(c * CH, CH), :] = ag_ref[(h + 1) % 2]

    return pl.pallas_call(
        body,
        out_shape=jax.ShapeDtypeStruct((M, N), jnp.bfloat16),
        in_specs=[
            pl.BlockSpec(memory_space=pltpu.VMEM),
            pl.BlockSpec(memory_space=pltpu.VMEM),
        ],
        out_specs=pl.BlockSpec(memory_space=pltpu.VMEM),
        scratch_shapes=[
            pltpu.VMEM((2, CH, N), jnp.bfloat16),
            pltpu.VMEM((2, CH, N), jnp.bfloat16),
            pltpu.SemaphoreType.DMA((N_DEV - 1,)),
            pltpu.SemaphoreType.DMA((N_DEV - 1,)),
            pltpu.SemaphoreType.DMA((N_DEV - 1,)),
            pltpu.SemaphoreType.DMA((N_DEV - 1,)),
        ],
        compiler_params=pltpu.CompilerParams(collective_id=0),
    )(A16, B16)


# baseline (device time: 705579 ns/iter reference)
import jax
import jax.numpy as jnp
from jax import lax
from jax.experimental import pallas as pl
from jax.experimental.pallas import tpu as pltpu

N_DEV = 4


def kernel(A, B):
    M = A.shape[0]
    N = B.shape[1]
    CH = M // N_DEV
    TJ = 1024

    A16 = A.astype(jnp.bfloat16)
    B16 = B.astype(jnp.bfloat16)

    def body(a_ref, b_ref, out_ref, comm_ref,
             rs_send, rs_recv, ag_send, ag_recv, copy_sem):
        my = lax.axis_index("i")
        right = lax.rem(my + 1, N_DEV)
        left = lax.rem(my + N_DEV - 1, N_DEV)

        barrier = pltpu.get_barrier_semaphore()
        for nbr in (left, right):
            pl.semaphore_signal(barrier, inc=1, device_id=(nbr,),
                                device_id_type=pl.DeviceIdType.MESH)
        pl.semaphore_wait(barrier, 2)

        def fill_chunk(c, slot):
            for j in range(N // TJ):
                col = pl.ds(j * TJ, TJ)
                p = jnp.dot(a_ref[pl.ds(c * CH, CH), :], b_ref[:, col],
                            preferred_element_type=jnp.float32)
                comm_ref[slot, :, col] = p.astype(jnp.bfloat16)

        def accum_chunk(c, slot):
            for j in range(N // TJ):
                col = pl.ds(j * TJ, TJ)
                p = jnp.dot(a_ref[pl.ds(c * CH, CH), :], b_ref[:, col],
                            preferred_element_type=jnp.float32)
                comm_ref[slot, :, col] = (
                    comm_ref[slot, :, col].astype(jnp.float32) + p
                ).astype(jnp.bfloat16)

        def store_chunk(c, slot):
            copy = pltpu.make_async_copy(
                comm_ref.at[slot],
                out_ref.at[pl.ds(c * CH, CH), :],
                copy_sem,
            )
            copy.start()
            copy.wait()

        fill_chunk(my, 0)
        for s in range(N_DEV - 1):
            rdma = pltpu.make_async_remote_copy(
                src_ref=comm_ref.at[s % 2],
                dst_ref=comm_ref.at[(s + 1) % 2],
                send_sem=rs_send.at[s],
                recv_sem=rs_recv.at[s],
                device_id=(right,),
                device_id_type=pl.DeviceIdType.MESH,
            )
            rdma.start()
            rdma.wait()
            c = lax.rem(my - s - 1 + N_DEV, N_DEV)
            accum_chunk(c, (s + 1) % 2)

        own = lax.rem(my + 1, N_DEV)
        store_chunk(own, 1)

        for h in range(N_DEV - 1):
            rdma = pltpu.make_async_remote_copy(
                src_ref=comm_ref.at[(h + 1) % 2],
                dst_ref=comm_ref.at[h % 2],
                send_sem=ag_send.at[h],
                recv_sem=ag_recv.at[h],
                device_id=(right,),
                device_id_type=pl.DeviceIdType.MESH,
            )
            rdma.start()
            rdma.wait()
            c = lax.rem(my - h + N_DEV, N_DEV)
            store_chunk(c, h % 2)

    return pl.pallas_call(
        body,
        out_shape=jax.ShapeDtypeStruct((M, N), jnp.bfloat16),
        in_specs=[
            pl.BlockSpec(memory_space=pltpu.VMEM),
            pl.BlockSpec(memory_space=pltpu.VMEM),
        ],
        out_specs=pl.BlockSpec(memory_space=pl.ANY),
        scratch_shapes=[
            pltpu.VMEM((2, CH, N), jnp.bfloat16),
            pltpu.SemaphoreType.DMA((N_DEV - 1,)),
            pltpu.SemaphoreType.DMA((N_DEV - 1,)),
            pltpu.SemaphoreType.DMA((N_DEV - 1,)),
            pltpu.SemaphoreType.DMA((N_DEV - 1,)),
            pltpu.SemaphoreType.DMA,
        ],
        compiler_params=pltpu.CompilerParams(
            collective_id=0,
            vmem_limit_bytes=60 * 1024 * 1024,
        ),
    )(A16, B16)


# device time: 377908 ns/iter; 1.8671x vs baseline; 1.8671x over previous
import jax
import jax.numpy as jnp
from jax import lax
from jax.experimental import pallas as pl
from jax.experimental.pallas import tpu as pltpu

N_DEV = 4


def kernel(A, B):
    M = A.shape[0]
    N = B.shape[1]
    CH = M // N_DEV
    H = N // 2
    TJ = 1024

    A16 = A.astype(jnp.bfloat16)
    B16 = B.astype(jnp.bfloat16)

    def body(a_ref, b_ref, out_ref, commR, commL, p_ref,
             rs_send, rs_recv, ag_send, ag_recv, copy_sems):
        my = lax.axis_index("i")
        right = lax.rem(my + 1, N_DEV)
        left = lax.rem(my + N_DEV - 1, N_DEV)

        barrier = pltpu.get_barrier_semaphore()
        for nbr in (left, right):
            pl.semaphore_signal(barrier, inc=1, device_id=(nbr,),
                                device_id_type=pl.DeviceIdType.MESH)
        pl.semaphore_wait(barrier, 2)

        def a_blk(c):
            return a_ref[pl.ds(c * CH, CH), :]

        def matmul_half(c, col0, store):
            for j in range(H // TJ):
                p = jnp.dot(a_blk(c), b_ref[:, pl.ds(col0 + j * TJ, TJ)],
                            preferred_element_type=jnp.float32)
                store(j, p.astype(jnp.bfloat16))

        def precompute(cR, cL):
            matmul_half(cR, 0,
                        lambda j, t: p_ref.__setitem__(
                            (slice(None), pl.ds(j * TJ, TJ)), t))
            matmul_half(cL, H,
                        lambda j, t: p_ref.__setitem__(
                            (slice(None), pl.ds(H + j * TJ, TJ)), t))

        def add_staged(comm, slot, p_col0):
            for j in range(H // TJ):
                col = pl.ds(j * TJ, TJ)
                pcol = pl.ds(p_col0 + j * TJ, TJ)
                comm[slot, :, col] = (
                    comm[slot, :, col].astype(jnp.float32)
                    + p_ref[:, pcol].astype(jnp.float32)
                ).astype(jnp.bfloat16)

        matmul_half(my, 0,
                    lambda j, t: commR.__setitem__(
                        (0, slice(None), pl.ds(j * TJ, TJ)), t))
        matmul_half(my, H,
                    lambda j, t: commL.__setitem__(
                        (0, slice(None), pl.ds(j * TJ, TJ)), t))

        for s in range(N_DEV - 1):
            rdmaR = pltpu.make_async_remote_copy(
                src_ref=commR.at[s % 2], dst_ref=commR.at[(s + 1) % 2],
                send_sem=rs_send.at[s, 0], recv_sem=rs_recv.at[s, 0],
                device_id=(right,), device_id_type=pl.DeviceIdType.MESH,
            )
            rdmaL = pltpu.make_async_remote_copy(
                src_ref=commL.at[s % 2], dst_ref=commL.at[(s + 1) % 2],
                send_sem=rs_send.at[s, 1], recv_sem=rs_recv.at[s, 1],
                device_id=(left,), device_id_type=pl.DeviceIdType.MESH,
            )
            rdmaR.start()
            rdmaL.start()
            cR = lax.rem(my - s - 1 + N_DEV, N_DEV)
            cL = lax.rem(my + s + 1, N_DEV)
            precompute(cR, cL)
            rdmaR.wait()
            add_staged(commR, (s + 1) % 2, 0)
            rdmaL.wait()
            add_staged(commL, (s + 1) % 2, H)

        copies = []

        def store_half(comm, slot, c, col0, sem_idx):
            cp = pltpu.make_async_copy(
                comm.at[slot],
                out_ref.at[pl.ds(c * CH, CH), pl.ds(col0, H)],
                copy_sems.at[sem_idx],
            )
            cp.start()
            copies.append(cp)

        store_half(commR, 1, lax.rem(my + 1, N_DEV), 0, 0)
        store_half(commL, 1, lax.rem(my + 3, N_DEV), H, 1)

        for h in range(N_DEV - 1):
            rdmaR = pltpu.make_async_remote_copy(
                src_ref=commR.at[(h + 1) % 2], dst_ref=commR.at[h % 2],
                send_sem=ag_send.at[h, 0], recv_sem=ag_recv.at[h, 0],
                device_id=(right,), device_id_type=pl.DeviceIdType.MESH,
            )
            rdmaL = pltpu.make_async_remote_copy(
                src_ref=commL.at[(h + 1) % 2], dst_ref=commL.at[h % 2],
                send_sem=ag_send.at[h, 1], recv_sem=ag_recv.at[h, 1],
                device_id=(left,), device_id_type=pl.DeviceIdType.MESH,
            )
            rdmaR.start()
            rdmaL.start()
            rdmaR.wait()
            store_half(commR, h % 2, lax.rem(my - h + N_DEV, N_DEV), 0,
                       2 + 2 * h)
            rdmaL.wait()
            store_half(commL, h % 2, lax.rem(my + h, N_DEV), H,
                       3 + 2 * h)

        for cp in copies:
            cp.wait()

    return pl.pallas_call(
        body,
        out_shape=jax.ShapeDtypeStruct((M, N), jnp.bfloat16),
        in_specs=[
            pl.BlockSpec(memory_space=pltpu.VMEM),
            pl.BlockSpec(memory_space=pltpu.VMEM),
        ],
        out_specs=pl.BlockSpec(memory_space=pl.ANY),
        scratch_shapes=[
            pltpu.VMEM((2, CH, H), jnp.bfloat16),
            pltpu.VMEM((2, CH, H), jnp.bfloat16),
            pltpu.VMEM((CH, N), jnp.bfloat16),
            pltpu.SemaphoreType.DMA((N_DEV - 1, 2)),
            pltpu.SemaphoreType.DMA((N_DEV - 1, 2)),
            pltpu.SemaphoreType.DMA((N_DEV - 1, 2)),
            pltpu.SemaphoreType.DMA((N_DEV - 1, 2)),
            pltpu.SemaphoreType.DMA((8,)),
        ],
        compiler_params=pltpu.CompilerParams(
            collective_id=0,
            vmem_limit_bytes=60 * 1024 * 1024,
        ),
    )(A16, B16)


# device time: 376000 ns/iter; 1.8765x vs baseline; 1.0051x over previous
import jax
import jax.numpy as jnp
from jax import lax
from jax.experimental import pallas as pl
from jax.experimental.pallas import tpu as pltpu

N_DEV = 4


def kernel(A, B):
    M = A.shape[0]
    N = B.shape[1]
    CH = M // N_DEV
    H = N // 2
    TJ = 1024

    A16 = A.astype(jnp.bfloat16)
    B16 = B.astype(jnp.bfloat16)

    def body(a_ref, b_ref, out_ref, commR, commL, p_ref,
             rs_send, rs_recv, ag_send, ag_recv, copy_sems):
        my = lax.axis_index("i")
        right = lax.rem(my + 1, N_DEV)
        left = lax.rem(my + N_DEV - 1, N_DEV)

        barrier = pltpu.get_barrier_semaphore()
        for nbr in (left, right):
            pl.semaphore_signal(barrier, inc=1, device_id=(nbr,),
                                device_id_type=pl.DeviceIdType.MESH)
        pl.semaphore_wait(barrier, 2)

        def a_blk(c):
            return a_ref[pl.ds(c * CH, CH), :]

        def matmul_half(c, col0, store):
            for j in range(H // TJ):
                p = jnp.dot(a_blk(c), b_ref[:, pl.ds(col0 + j * TJ, TJ)],
                            preferred_element_type=jnp.float32)
                store(j, p.astype(jnp.bfloat16))

        def precompute(cR, cL):
            matmul_half(cR, 0,
                        lambda j, t: p_ref.__setitem__(
                            (slice(None), pl.ds(j * TJ, TJ)), t))
            matmul_half(cL, H,
                        lambda j, t: p_ref.__setitem__(
                            (slice(None), pl.ds(H + j * TJ, TJ)), t))

        def add_staged(comm, slot, p_col0):
            for j in range(H // TJ):
                col = pl.ds(j * TJ, TJ)
                pcol = pl.ds(p_col0 + j * TJ, TJ)
                comm[slot, :, col] = (
                    comm[slot, :, col].astype(jnp.float32)
                    + p_ref[:, pcol].astype(jnp.float32)
                ).astype(jnp.bfloat16)

        matmul_half(my, 0,
                    lambda j, t: commR.__setitem__(
                        (0, slice(None), pl.ds(j * TJ, TJ)), t))
        matmul_half(my, H,
                    lambda j, t: commL.__setitem__(
                        (0, slice(None), pl.ds(j * TJ, TJ)), t))

        rsR = [
            pltpu.make_async_remote_copy(
                src_ref=commR.at[s % 2], dst_ref=commR.at[(s + 1) % 2],
                send_sem=rs_send.at[s, 0], recv_sem=rs_recv.at[s, 0],
                device_id=(right,), device_id_type=pl.DeviceIdType.MESH,
            )
            for s in range(N_DEV - 1)
        ]
        rsL = [
            pltpu.make_async_remote_copy(
                src_ref=commL.at[s % 2], dst_ref=commL.at[(s + 1) % 2],
                send_sem=rs_send.at[s, 1], recv_sem=rs_recv.at[s, 1],
                device_id=(left,), device_id_type=pl.DeviceIdType.MESH,
            )
            for s in range(N_DEV - 1)
        ]
        rsR[0].start()
        rsL[0].start()
        precompute(lax.rem(my - 1 + N_DEV, N_DEV), lax.rem(my + 1, N_DEV))
        for s in range(N_DEV - 1):
            rsR[s].wait()
            add_staged(commR, (s + 1) % 2, 0)
            if s < N_DEV - 2:
                rsR[s + 1].start()
            rsL[s].wait()
            add_staged(commL, (s + 1) % 2, H)
            if s < N_DEV - 2:
                rsL[s + 1].start()
                precompute(lax.rem(my - s - 2 + N_DEV, N_DEV),
                           lax.rem(my + s + 2, N_DEV))

        copies = []

        def store_half(comm, slot, c, col0, sem_idx):
            cp = pltpu.make_async_copy(
                comm.at[slot],
                out_ref.at[pl.ds(c * CH, CH), pl.ds(col0, H)],
                copy_sems.at[sem_idx],
            )
            cp.start()
            copies.append(cp)

        store_half(commR, 1, lax.rem(my + 1, N_DEV), 0, 0)
        store_half(commL, 1, lax.rem(my + 3, N_DEV), H, 1)

        agR = [
            pltpu.make_async_remote_copy(
                src_ref=commR.at[(h + 1) % 2], dst_ref=commR.at[h % 2],
                send_sem=ag_send.at[h, 0], recv_sem=ag_recv.at[h, 0],
                device_id=(right,), device_id_type=pl.DeviceIdType.MESH,
            )
            for h in range(N_DEV - 1)
        ]
        agL = [
            pltpu.make_async_remote_copy(
                src_ref=commL.at[(h + 1) % 2], dst_ref=commL.at[h % 2],
                send_sem=ag_send.at[h, 1], recv_sem=ag_recv.at[h, 1],
                device_id=(left,), device_id_type=pl.DeviceIdType.MESH,
            )
            for h in range(N_DEV - 1)
        ]
        agR[0].start()
        agL[0].start()
        for h in range(N_DEV - 1):
            agR[h].wait()
            if h < N_DEV - 2:
                agR[h + 1].start()
            store_half(commR, h % 2, lax.rem(my - h + N_DEV, N_DEV), 0,
                       2 + 2 * h)
            agL[h].wait()
            if h < N_DEV - 2:
                agL[h + 1].start()
            store_half(commL, h % 2, lax.rem(my + h, N_DEV), H,
                       3 + 2 * h)

        for cp in copies:
            cp.wait()

    return pl.pallas_call(
        body,
        out_shape=jax.ShapeDtypeStruct((M, N), jnp.bfloat16),
        in_specs=[
            pl.BlockSpec(memory_space=pltpu.VMEM),
            pl.BlockSpec(memory_space=pltpu.VMEM),
        ],
        out_specs=pl.BlockSpec(memory_space=pl.ANY),
        scratch_shapes=[
            pltpu.VMEM((2, CH, H), jnp.bfloat16),
            pltpu.VMEM((2, CH, H), jnp.bfloat16),
            pltpu.VMEM((CH, N), jnp.bfloat16),
            pltpu.SemaphoreType.DMA((N_DEV - 1, 2)),
            pltpu.SemaphoreType.DMA((N_DEV - 1, 2)),
            pltpu.SemaphoreType.DMA((N_DEV - 1, 2)),
            pltpu.SemaphoreType.DMA((N_DEV - 1, 2)),
            pltpu.SemaphoreType.DMA((8,)),
        ],
        compiler_params=pltpu.CompilerParams(
            collective_id=0,
            vmem_limit_bytes=60 * 1024 * 1024,
        ),
    )(A16, B16)


# device time: 357659 ns/iter; 1.9728x vs baseline; 1.0513x over previous
import jax
import jax.numpy as jnp
from jax import lax
from jax.experimental import pallas as pl
from jax.experimental.pallas import tpu as pltpu

N_DEV = 4


def kernel(A, B):
    M = A.shape[0]
    N = B.shape[1]
    CH = M // N_DEV
    H = N // 2
    TJ = 1024
    NSUB = H // TJ

    A16 = A.astype(jnp.bfloat16)
    B16 = B.astype(jnp.bfloat16)

    def body(a_ref, b_ref, out_ref, commR, commL, p_ref,
             rs_send, rs_recv, ag_send, ag_recv, copy_sems):
        my = lax.axis_index("i")
        right = lax.rem(my + 1, N_DEV)
        left = lax.rem(my + N_DEV - 1, N_DEV)

        barrier = pltpu.get_barrier_semaphore()
        for nbr in (left, right):
            pl.semaphore_signal(barrier, inc=1, device_id=(nbr,),
                                device_id_type=pl.DeviceIdType.MESH)
        pl.semaphore_wait(barrier, 2)

        def a_blk(c):
            return a_ref[pl.ds(c * CH, CH), :]

        def mm_tile(c, col0):
            return jnp.dot(a_blk(c), b_ref[:, pl.ds(col0, TJ)],
                           preferred_element_type=jnp.float32
                           ).astype(jnp.bfloat16)

        def precompute(cR, cL):
            for k in range(NSUB):
                p_ref[:, pl.ds(k * TJ, TJ)] = mm_tile(cR, k * TJ)
            for k in range(NSUB):
                p_ref[:, pl.ds(H + k * TJ, TJ)] = mm_tile(cL, H + k * TJ)

        def add_sub(comm, slot, k, p_col0):
            col = pl.ds(k * TJ, TJ)
            pcol = pl.ds(p_col0 + k * TJ, TJ)
            comm[slot, :, col] = (
                comm[slot, :, col].astype(jnp.float32)
                + p_ref[:, pcol].astype(jnp.float32)
            ).astype(jnp.bfloat16)

        def rs_desc(s, k, comm, sub, dev):
            return pltpu.make_async_remote_copy(
                src_ref=comm.at[s % 2, :, pl.ds(k * TJ, TJ)],
                dst_ref=comm.at[(s + 1) % 2, :, pl.ds(k * TJ, TJ)],
                send_sem=rs_send.at[s, sub, k],
                recv_sem=rs_recv.at[s, sub, k],
                device_id=(dev,), device_id_type=pl.DeviceIdType.MESH,
            )

        rsR = [[rs_desc(s, k, commR, 0, right) for k in range(NSUB)]
               for s in range(N_DEV - 1)]
        rsL = [[rs_desc(s, k, commL, 1, left) for k in range(NSUB)]
               for s in range(N_DEV - 1)]

        agR = [
            pltpu.make_async_remote_copy(
                src_ref=commR.at[(h + 1) % 2], dst_ref=commR.at[h % 2],
                send_sem=ag_send.at[h, 0], recv_sem=ag_recv.at[h, 0],
                device_id=(right,), device_id_type=pl.DeviceIdType.MESH,
            )
            for h in range(N_DEV - 1)
        ]
        agL = [
            pltpu.make_async_remote_copy(
                src_ref=commL.at[(h + 1) % 2], dst_ref=commL.at[h % 2],
                send_sem=ag_send.at[h, 1], recv_sem=ag_recv.at[h, 1],
                device_id=(left,), device_id_type=pl.DeviceIdType.MESH,
            )
            for h in range(N_DEV - 1)
        ]

        copies = []

        def store_half(comm, slot, c, col0, sem_idx):
            cp = pltpu.make_async_copy(
                comm.at[slot],
                out_ref.at[pl.ds(c * CH, CH), pl.ds(col0, H)],
                copy_sems.at[sem_idx],
            )
            cp.start()
            copies.append(cp)

        for k in range(NSUB):
            commR[0, :, pl.ds(k * TJ, TJ)] = mm_tile(my, k * TJ)
            rsR[0][k].start()
            commL[0, :, pl.ds(k * TJ, TJ)] = mm_tile(my, H + k * TJ)
            rsL[0][k].start()

        precompute(lax.rem(my - 1 + N_DEV, N_DEV), lax.rem(my + 1, N_DEV))

        for s in range(N_DEV - 1):
            last = s == N_DEV - 2
            for k in range(NSUB):
                rsR[s][k].wait()
                add_sub(commR, (s + 1) % 2, k, 0)
                if not last:
                    rsR[s + 1][k].start()
                elif k == NSUB - 1:
                    store_half(commR, 1, lax.rem(my + 1, N_DEV), 0, 0)
                    agR[0].start()
                rsL[s][k].wait()
                add_sub(commL, (s + 1) % 2, k, H)
                if not last:
                    rsL[s + 1][k].start()
                elif k == NSUB - 1:
                    store_half(commL, 1, lax.rem(my + 3, N_DEV), H, 1)
                    agL[0].start()
            if not last:
                precompute(lax.rem(my - s - 2 + N_DEV, N_DEV),
                           lax.rem(my + s + 2, N_DEV))

        for h in range(N_DEV - 1):
            agR[h].wait()
            if h < N_DEV - 2:
                agR[h + 1].start()
            store_half(commR, h % 2, lax.rem(my - h + N_DEV, N_DEV), 0,
                       2 + 2 * h)
            agL[h].wait()
            if h < N_DEV - 2:
                agL[h + 1].start()
            store_half(commL, h % 2, lax.rem(my + h, N_DEV), H,
                       3 + 2 * h)

        for cp in copies:
            cp.wait()

    return pl.pallas_call(
        body,
        out_shape=jax.ShapeDtypeStruct((M, N), jnp.bfloat16),
        in_specs=[
            pl.BlockSpec(memory_space=pltpu.VMEM),
            pl.BlockSpec(memory_space=pltpu.VMEM),
        ],
        out_specs=pl.BlockSpec(memory_space=pl.ANY),
        scratch_shapes=[
            pltpu.VMEM((2, CH, H), jnp.bfloat16),
            pltpu.VMEM((2, CH, H), jnp.bfloat16),
            pltpu.VMEM((CH, N), jnp.bfloat16),
            pltpu.SemaphoreType.DMA((N_DEV - 1, 2, 2)),
            pltpu.SemaphoreType.DMA((N_DEV - 1, 2, 2)),
            pltpu.SemaphoreType.DMA((N_DEV - 1, 2)),
            pltpu.SemaphoreType.DMA((N_DEV - 1, 2)),
            pltpu.SemaphoreType.DMA((8,)),
        ],
        compiler_params=pltpu.CompilerParams(
            collective_id=0,
            vmem_limit_bytes=60 * 1024 * 1024,
        ),
    )(A16, B16)


# device time: 352383 ns/iter; 2.0023x vs baseline; 1.0150x over previous
import jax
import jax.numpy as jnp
from jax import lax
from jax.experimental import pallas as pl
from jax.experimental.pallas import tpu as pltpu

N_DEV = 4


def kernel(A, B):
    M = A.shape[0]
    N = B.shape[1]
    CH = M // N_DEV
    H = N // 2
    TJ = 1024
    NSUB = H // TJ

    A16 = A.astype(jnp.bfloat16)
    B16 = B.astype(jnp.bfloat16)

    def body(a_ref, b_ref, out_ref, commR, commL, p_ref,
             rs_send, rs_recv, ag_send, ag_recv, copy_sems):
        my = lax.axis_index("i")
        right = lax.rem(my + 1, N_DEV)
        left = lax.rem(my + N_DEV - 1, N_DEV)

        barrier = pltpu.get_barrier_semaphore()
        for nbr in (left, right):
            pl.semaphore_signal(barrier, inc=1, device_id=(nbr,),
                                device_id_type=pl.DeviceIdType.MESH)
        pl.semaphore_wait(barrier, 2)

        def a_blk(c):
            return a_ref[pl.ds(c * CH, CH), :]

        def mm_tile(c, col0):
            return jnp.dot(a_blk(c), b_ref[:, pl.ds(col0, TJ)],
                           preferred_element_type=jnp.float32
                           ).astype(jnp.bfloat16)

        def precompute(cR, cL):
            for k in range(NSUB):
                p_ref[:, pl.ds(k * TJ, TJ)] = mm_tile(cR, k * TJ)
            for k in range(NSUB):
                p_ref[:, pl.ds(H + k * TJ, TJ)] = mm_tile(cL, H + k * TJ)

        def add_sub(comm, slot, k, p_col0):
            col = pl.ds(k * TJ, TJ)
            pcol = pl.ds(p_col0 + k * TJ, TJ)
            comm[slot, :, col] = (
                comm[slot, :, col].astype(jnp.float32)
                + p_ref[:, pcol].astype(jnp.float32)
            ).astype(jnp.bfloat16)

        def rs_desc(s, k, comm, sub, dev):
            return pltpu.make_async_remote_copy(
                src_ref=comm.at[s % 2, :, pl.ds(k * TJ, TJ)],
                dst_ref=comm.at[(s + 1) % 2, :, pl.ds(k * TJ, TJ)],
                send_sem=rs_send.at[s, sub, k],
                recv_sem=rs_recv.at[s, sub, k],
                device_id=(dev,), device_id_type=pl.DeviceIdType.MESH,
            )

        rsR = [[rs_desc(s, k, commR, 0, right) for k in range(NSUB)]
               for s in range(N_DEV - 1)]
        rsL = [[rs_desc(s, k, commL, 1, left) for k in range(NSUB)]
               for s in range(N_DEV - 1)]

        def ag_desc(h, k, comm, sub, dev):
            return pltpu.make_async_remote_copy(
                src_ref=comm.at[(h + 1) % 2, :, pl.ds(k * TJ, TJ)],
                dst_ref=comm.at[h % 2, :, pl.ds(k * TJ, TJ)],
                send_sem=ag_send.at[h, sub, k],
                recv_sem=ag_recv.at[h, sub, k],
                device_id=(dev,), device_id_type=pl.DeviceIdType.MESH,
            )

        agR = [[ag_desc(h, k, commR, 0, right) for k in range(NSUB)]
               for h in range(N_DEV - 1)]
        agL = [[ag_desc(h, k, commL, 1, left) for k in range(NSUB)]
               for h in range(N_DEV - 1)]

        copies = []

        def store_half(comm, slot, c, col0, sem_idx):
            cp = pltpu.make_async_copy(
                comm.at[slot],
                out_ref.at[pl.ds(c * CH, CH), pl.ds(col0, H)],
                copy_sems.at[sem_idx],
            )
            cp.start()
            copies.append(cp)

        for k in range(NSUB):
            commR[0, :, pl.ds(k * TJ, TJ)] = mm_tile(my, k * TJ)
            rsR[0][k].start()
            commL[0, :, pl.ds(k * TJ, TJ)] = mm_tile(my, H + k * TJ)
            rsL[0][k].start()

        precompute(lax.rem(my - 1 + N_DEV, N_DEV), lax.rem(my + 1, N_DEV))

        for s in range(N_DEV - 1):
            last = s == N_DEV - 2
            for k in range(NSUB):
                rsR[s][k].wait()
                add_sub(commR, (s + 1) % 2, k, 0)
                if not last:
                    rsR[s + 1][k].start()
                else:
                    agR[0][k].start()
                    if k == NSUB - 1:
                        store_half(commR, 1, lax.rem(my + 1, N_DEV), 0, 0)
                rsL[s][k].wait()
                add_sub(commL, (s + 1) % 2, k, H)
                if not last:
                    rsL[s + 1][k].start()
                else:
                    agL[0][k].start()
                    if k == NSUB - 1:
                        store_half(commL, 1, lax.rem(my + 3, N_DEV), H, 1)
            if not last:
                precompute(lax.rem(my - s - 2 + N_DEV, N_DEV),
                           lax.rem(my + s + 2, N_DEV))

        for h in range(N_DEV - 1):
            for k in range(NSUB):
                agR[h][k].wait()
                if h < N_DEV - 2:
                    agR[h + 1][k].start()
                if k == NSUB - 1:
                    store_half(commR, h % 2, lax.rem(my - h + N_DEV, N_DEV),
                               0, 2 + 2 * h)
                agL[h][k].wait()
                if h < N_DEV - 2:
                    agL[h + 1][k].start()
                if k == NSUB - 1:
                    store_half(commL, h % 2, lax.rem(my + h, N_DEV),
                               H, 3 + 2 * h)

        for cp in copies:
            cp.wait()

    return pl.pallas_call(
        body,
        out_shape=jax.ShapeDtypeStruct((M, N), jnp.bfloat16),
        in_specs=[
            pl.BlockSpec(memory_space=pltpu.VMEM),
            pl.BlockSpec(memory_space=pltpu.VMEM),
        ],
        out_specs=pl.BlockSpec(memory_space=pl.ANY),
        scratch_shapes=[
            pltpu.VMEM((2, CH, H), jnp.bfloat16),
            pltpu.VMEM((2, CH, H), jnp.bfloat16),
            pltpu.VMEM((CH, N), jnp.bfloat16),
            pltpu.SemaphoreType.DMA((N_DEV - 1, 2, 2)),
            pltpu.SemaphoreType.DMA((N_DEV - 1, 2, 2)),
            pltpu.SemaphoreType.DMA((N_DEV - 1, 2, 2)),
            pltpu.SemaphoreType.DMA((N_DEV - 1, 2, 2)),
            pltpu.SemaphoreType.DMA((8,)),
        ],
        compiler_params=pltpu.CompilerParams(
            collective_id=0,
            vmem_limit_bytes=60 * 1024 * 1024,
        ),
    )(A16, B16)
